# baseline (speedup 1.0000x reference)
import functools

import jax
import jax.numpy as jnp
from jax import lax
from jax.experimental import pallas as pl
from jax.experimental.pallas import tpu as pltpu
from jax.experimental.pallas import tpu_sc as plsc

N = 100000
D = 512
G = 512
C = 80
NC = 2
NS = 16
NW = NC * NS
GPT = G // NW
DV = D // 16
SB = 4000
NSB = N // SB
NB = NW + 16

_mesh = plsc.VectorSubcoreMesh(core_axis_name="c", subcore_axis_name="s")
_Z = None


def _lane_sum(v):
    iota = lax.iota(jnp.int32, 16)
    for shift in (8, 4, 2, 1):
        idx = jnp.bitwise_and(iota + shift, 15)
        v = v + jnp.take_along_axis(v, idx, 0, mode="promise_in_bounds")
    return v


def _row_bounds(batch_hbm, sbufs, sems, wid):
    lo_t = GPT * wid
    hi_t = lo_t + GPT
    clo = jnp.zeros((16,), jnp.int32)
    chi = jnp.zeros((16,), jnp.int32)
    pltpu.async_copy(batch_hbm.at[pl.ds(0, SB)], sbufs[0], sems[0])
    for c in range(NSB):
        b = c & 1
        pltpu.make_async_copy(
            batch_hbm.at[pl.ds(0, SB)], sbufs[b], sems[b]).wait()
        if c + 1 < NSB:
            pltpu.async_copy(
                batch_hbm.at[pl.ds((c + 1) * SB, SB)],
                sbufs[1 - b], sems[1 - b])
        sbuf = sbufs[b]

        def body(j, carry):
            cl, ch = carry
            bv = sbuf[pl.ds(j * 16, 16)]
            cl = cl + jnp.where(bv < lo_t, 1, 0)
            ch = ch + jnp.where(bv < hi_t, 1, 0)
            return (cl, ch)

        clo, chi = lax.fori_loop(0, SB // 16, body, (clo, chi))
    return _lane_sum(clo)[0], _lane_sum(chi)[0]


def _run_end(ib, i, g, i_hi):
    iota = lax.iota(jnp.int32, 16)

    def cond_fn(c):
        return c[1] < 0

    def body_fn(c):
        j = c[0]
        bv = ib[pl.ds(j, 16)]
        m = jnp.logical_and(bv == g, (j + iota) < i_hi)
        cnt = plsc.all_reduce_population_count(m)[0]
        end = jnp.where(cnt < 16, j + cnt, -1)
        return (j + 16, end)

    return lax.while_loop(cond_fn, body_fn, (i, jnp.int32(-1)))[1]


def _issue(x_hbm, batch_hbm, xb, ib, sx, sb, base):
    pltpu.async_copy(x_hbm.at[pl.ds(base, C)], xb, sx)
    pltpu.async_copy(batch_hbm.at[pl.ds(base, C)], ib.at[pl.ds(0, C)], sb)


def _wait(x_hbm, batch_hbm, xb, ib, sx, sb):
    pltpu.make_async_copy(x_hbm.at[pl.ds(0, C)], xb, sx).wait()
    pltpu.make_async_copy(
        batch_hbm.at[pl.ds(0, C)], ib.at[pl.ds(0, C)], sb).wait()


def _chunk_loop(x_hbm, batch_hbm, xbufs, ibufs, sems, lo, hi, make_row_body,
                init, make_chunk_post=None):
    start0 = (lo // 8) * 8
    nch = (hi - start0 + C - 1) // C
    nchm1 = jnp.maximum(nch - 1, 0)

    def cbase(k):
        return jnp.minimum(start0 + jnp.minimum(k, nchm1) * C, N - C)

    _issue(x_hbm, batch_hbm, xbufs[0], ibufs[0], sems[0], sems[1], cbase(0))
    _issue(x_hbm, batch_hbm, xbufs[1], ibufs[1], sems[2], sems[3], cbase(1))

    def pair_body(p, carry):
        for half in (0, 1):
            k = 2 * p + half
            xb, ib = xbufs[half], ibufs[half]
            sx, sb = sems[2 * half], sems[2 * half + 1]
            _wait(x_hbm, batch_hbm, xb, ib, sx, sb)
            base = cbase(k)
            i_lo = jnp.maximum(jnp.maximum(lo, start0 + k * C) - base, 0)
            i_hi = jnp.clip(hi - base, 0, C)
            i_hi = jnp.where(k < nch, i_hi, 0)
            carry = make_row_body(xb, ib)(i_lo, i_hi, carry)
            if make_chunk_post is not None:
                carry = make_chunk_post(xb, ib)(i_lo, i_hi, carry)
            _issue(x_hbm, batch_hbm, xb, ib, sx, sb, cbase(k + 2))
        return carry

    carry = lax.fori_loop(0, (nch + 1) // 2, pair_body, init)
    for half in (0, 1):
        _wait(x_hbm, batch_hbm, xbufs[half], ibufs[half],
              sems[2 * half], sems[2 * half + 1])
    return carry


@functools.partial(
    pl.kernel,
    mesh=_mesh,
    out_type=[
        jax.ShapeDtypeStruct((G * (D + 16),), jnp.float32),
        jax.ShapeDtypeStruct((NB * 16,), jnp.int32),
    ],
    scratch_types=[
        pltpu.VMEM((C, D), jnp.float32),
        pltpu.VMEM((C, D), jnp.float32),
        pltpu.VMEM((C + 16,), jnp.int32),
        pltpu.VMEM((C + 16,), jnp.int32),
        pltpu.VMEM((SB,), jnp.int32),
        pltpu.VMEM((SB,), jnp.int32),
        pltpu.VMEM((GPT * (D + 16),), jnp.float32),
    ] + [pltpu.SemaphoreType.DMA] * 6,
    compiler_params=pltpu.CompilerParams(needs_layout_passes=False),
)
def _sc_pass1(x_hbm, batch_hbm, sums_out, bounds_out,
              xb0, xb1, ib0, ib1, sb0, sb1, acc,
              s0, s1, s2, s3, s4, s5):
    cid = lax.axis_index("c")
    sid = lax.axis_index("s")
    wid = sid * NC + cid
    g0 = GPT * wid

    z = jnp.zeros((16,), jnp.float32)
    for v in range(GPT * (DV + 1)):
        acc[pl.ds(v * 16, 16)] = z

    lo, hi = _row_bounds(batch_hbm, (sb0, sb1), (s4, s5), wid)
    one = jnp.ones((16,), jnp.float32)

    def make_row_body(xb, ib):
        def proc(i_lo, i_hi, carry):
            def cond_fn(c):
                return c[0] < i_hi

            def body_fn(c):
                i, gprev = c[0], c[1]
                g = ib[pl.ds(i, 16)][0]

                def flush_br(ops):
                    o = (gprev - g0) * (D + 16)
                    for j in range(DV):
                        acc[pl.ds(o + j * 16, 16)] = ops[1 + j]
                    acc[pl.ds(o + D, 16)] = ops[0]
                    return (z,) * (DV + 1)

                def keep_br(ops):
                    return ops

                cur = lax.cond(
                    jnp.logical_and(g != gprev, gprev >= 0),
                    flush_br, keep_br, c[2:])
                end = _run_end(ib, i, g, i_hi)

                @plsc.parallel_loop(i, end, unroll=4, carry=cur)
                def cur(ii, ic):
                    new = tuple(
                        a + xb[ii, pl.ds(j * 16, 16)]
                        for j, a in enumerate(ic[1:]))
                    return (ic[0] + one,) + new

                return (end, g) + cur

            return lax.while_loop(cond_fn, body_fn, (i_lo,) + carry)[1:]

        return proc

    init = (jnp.int32(-1), z) + (z,) * DV
    carry = _chunk_loop(x_hbm, batch_hbm, (xb0, xb1), (ib0, ib1),
                        (s0, s1, s2, s3), lo, hi, make_row_body, init)
    gprev, cnt = carry[0], carry[1]
    accs = carry[2:]

    @pl.when(gprev >= 0)
    def _final_flush():
        o = (gprev - g0) * (D + 16)
        for j in range(DV):
            acc[pl.ds(o + j * 16, 16)] = accs[j]
        acc[pl.ds(o + D, 16)] = cnt

    pltpu.sync_copy(
        acc, sums_out.at[pl.ds(wid * GPT * (D + 16), GPT * (D + 16))])
    ib0[pl.ds(0, 16)] = jnp.full((16,), lo, jnp.int32)
    pltpu.sync_copy(ib0.at[pl.ds(0, 16)], bounds_out.at[pl.ds(wid * 16, 16)])

    @pl.when(wid == NW - 1)
    def _last_bound():
        ib1[pl.ds(0, 16)] = jnp.full((16,), hi, jnp.int32)
        pltpu.sync_copy(ib1.at[pl.ds(0, 16)],
                        bounds_out.at[pl.ds(NW * 16, 16)])


def _tc_ctx(sums_ref, w_ref, b_ref, ctx_ref):
    sums = sums_ref[:, :D]
    counts = jnp.maximum(sums_ref[:, D:D + 1], 1.0)
    mean = sums / counts
    ctx = lax.dot_general(mean, w_ref[...], (((1,), (1,)), ((), ())),
                          preferred_element_type=jnp.float32)
    ctx_ref[...] = jnp.tanh(ctx + b_ref[0, :][None, :])


@functools.partial(
    pl.kernel,
    mesh=_mesh,
    out_type=jax.ShapeDtypeStruct((G * D,), jnp.float32),
    scratch_types=[
        pltpu.VMEM((C, D), jnp.float32),
        pltpu.VMEM((C, D), jnp.float32),
        pltpu.VMEM((C + 16,), jnp.int32),
        pltpu.VMEM((C + 16,), jnp.int32),
        pltpu.VMEM((32,), jnp.int32),
        pltpu.VMEM((GPT, D), jnp.float32),
        pltpu.VMEM((GPT * D,), jnp.float32),
        pltpu.VMEM((C * 16,), jnp.float32),
        pltpu.VMEM((C,), jnp.float32),
    ] + [pltpu.SemaphoreType.DMA] * 4,
    compiler_params=pltpu.CompilerParams(needs_layout_passes=False),
)
def _sc_pass2(x_hbm, batch_hbm, ctx_hbm, bounds_hbm, out_hbm,
              xb0, xb1, ib0, ib1, bb, ctxb, acc, dbuf, svbuf,
              s0, s1, s2, s3):
    cid = lax.axis_index("c")
    sid = lax.axis_index("s")
    wid = sid * NC + cid
    g0 = GPT * wid

    z = jnp.zeros((16,), jnp.float32)
    for v in range(GPT * DV):
        acc[pl.ds(v * 16, 16)] = z

    pltpu.sync_copy(ctx_hbm.at[pl.ds(g0, GPT)], ctxb)
    pltpu.sync_copy(bounds_hbm.at[pl.ds(wid * 16, 32)], bb)
    lo = bb[pl.ds(0, 16)][0]
    hi = bb[pl.ds(16, 16)][0]
    iota = lax.iota(jnp.int32, 16)

    def make_row_body(xb, ib):
        def proc(i_lo, i_hi, carry):
            def cond_fn(c):
                return c[0] < i_hi

            def body_fn(c):
                i = c[0]
                g = ib[pl.ds(i, 16)][0]
                gi = g - g0
                ctxv = [ctxb[gi, pl.ds(j * 16, 16)] for j in range(DV)]
                end = _run_end(ib, i, g, i_hi)

                @plsc.parallel_loop(i, end, unroll=4)
                def inner(ii):
                    d = [z, z, z, z]
                    for j in range(DV):
                        d[j & 3] = d[j & 3] + xb[ii, pl.ds(j * 16, 16)] * ctxv[j]
                    dbuf[pl.ds(ii * 16, 16)] = (d[0] + d[1]) + (d[2] + d[3])

                return (end,)

            lax.while_loop(cond_fn, body_fn, (i_lo,))
            return carry

        return proc

    def make_chunk_post(xb, ib):
        def chunk_post(i_lo, i_hi, carry):
            for r0 in range(0, C, 16):
                s = z
                for cc in range(16):
                    s = s + plsc.load_gather(
                        dbuf, [(r0 + iota) * 16 + cc])
                svbuf[pl.ds(r0, 16)] = 1.0 / (1.0 + jnp.exp(-s))

            def cond_fn(c):
                return c[0] < i_hi

            def body_fn(c):
                i, gc = c[0], c[1]
                g = ib[pl.ds(i, 16)][0]

                def flush_br(ops):
                    o = (gc - g0) * D
                    for j in range(DV):
                        acc[pl.ds(o + j * 16, 16)] = ops[j]
                    return (z,) * DV

                def keep_br(ops):
                    return ops

                cur = lax.cond(
                    jnp.logical_and(g != gc, gc >= 0),
                    flush_br, keep_br, c[2:])
                end = _run_end(ib, i, g, i_hi)

                @plsc.parallel_loop(i, end, unroll=4, carry=cur)
                def cur(ii, ic):
                    sv = plsc.load_gather(
                        svbuf, [jnp.full((16,), ii, jnp.int32)])
                    return tuple(
                        a + sv * xb[ii, pl.ds(j * 16, 16)]
                        for j, a in enumerate(ic))

                return (end, g) + cur

            return lax.while_loop(cond_fn, body_fn, (i_lo,) + carry)[1:]

        return chunk_post

    init = (jnp.int32(-1),) + (z,) * DV
    carry = _chunk_loop(x_hbm, batch_hbm, (xb0, xb1), (ib0, ib1),
                        (s0, s1, s2, s3), lo, hi, make_row_body,
                        init, make_chunk_post)
    gprev = carry[0]
    accs = carry[1:]

    @pl.when(gprev >= 0)
    def _final_flush():
        o = (gprev - g0) * D
        for j in range(DV):
            acc[pl.ds(o + j * 16, 16)] = accs[j]

    pltpu.sync_copy(acc, out_hbm.at[pl.ds(wid * GPT * D, GPT * D)])


@jax.jit
def kernel(x, batch, W, b):
    batch = batch.astype(jnp.int32)
    sums_cnt, bounds = _sc_pass1(x, batch)
    ctx = pl.pallas_call(
        _tc_ctx,
        out_shape=jax.ShapeDtypeStruct((G, D), jnp.float32),
    )(sums_cnt.reshape(G, D + 16), W, b.reshape(1, D))
    return _sc_pass2(x, batch, ctx, bounds).reshape(G, D)


# speedup vs baseline: 1.3339x; 1.3339x over previous
import functools

import jax
import jax.numpy as jnp
from jax import lax
from jax.experimental import pallas as pl
from jax.experimental.pallas import tpu as pltpu
from jax.experimental.pallas import tpu_sc as plsc

N = 100000
D = 512
G = 512
C = 80
NC = 2
NS = 16
NW = NC * NS
GPT = G // NW
DV = D // 16
SB = 4000
NSB = N // SB
NB = NW + 16

_mesh = plsc.VectorSubcoreMesh(core_axis_name="c", subcore_axis_name="s")
_Z = None


def _lane_sum(v):
    iota = lax.iota(jnp.int32, 16)
    for shift in (8, 4, 2, 1):
        idx = jnp.bitwise_and(iota + shift, 15)
        v = v + jnp.take_along_axis(v, idx, 0, mode="promise_in_bounds")
    return v


def _row_bounds(batch_hbm, sbufs, sems, wid):
    lo_t = GPT * wid
    hi_t = lo_t + GPT
    clo = jnp.zeros((16,), jnp.int32)
    chi = jnp.zeros((16,), jnp.int32)
    pltpu.async_copy(batch_hbm.at[pl.ds(0, SB)], sbufs[0], sems[0])
    for c in range(NSB):
        b = c & 1
        pltpu.make_async_copy(
            batch_hbm.at[pl.ds(0, SB)], sbufs[b], sems[b]).wait()
        if c + 1 < NSB:
            pltpu.async_copy(
                batch_hbm.at[pl.ds((c + 1) * SB, SB)],
                sbufs[1 - b], sems[1 - b])
        sbuf = sbufs[b]

        def body(j, carry):
            cl, ch = carry
            bv = sbuf[pl.ds(j * 16, 16)]
            cl = cl + jnp.where(bv < lo_t, 1, 0)
            ch = ch + jnp.where(bv < hi_t, 1, 0)
            return (cl, ch)

        clo, chi = lax.fori_loop(0, SB // 16, body, (clo, chi))
    return _lane_sum(clo)[0], _lane_sum(chi)[0]


def _run_end(ib, i, g, i_hi):
    iota = lax.iota(jnp.int32, 16)

    def cond_fn(c):
        return c[1] < 0

    def body_fn(c):
        j = c[0]
        bv = ib[pl.ds(j, 16)]
        m = jnp.logical_and(bv == g, (j + iota) < i_hi)
        cnt = plsc.all_reduce_population_count(m)[0]
        end = jnp.where(cnt < 16, j + cnt, -1)
        return (j + 16, end)

    return lax.while_loop(cond_fn, body_fn, (i, jnp.int32(-1)))[1]


def _issue(x_hbm, batch_hbm, xb, ib, sx, sb, base):
    pltpu.async_copy(x_hbm.at[pl.ds(base, C)], xb, sx)
    pltpu.async_copy(batch_hbm.at[pl.ds(base, C)], ib.at[pl.ds(0, C)], sb)


def _wait(x_hbm, batch_hbm, xb, ib, sx, sb):
    pltpu.make_async_copy(x_hbm.at[pl.ds(0, C)], xb, sx).wait()
    pltpu.make_async_copy(
        batch_hbm.at[pl.ds(0, C)], ib.at[pl.ds(0, C)], sb).wait()


def _chunk_loop(x_hbm, batch_hbm, xbufs, ibufs, sems, lo, hi, make_row_body,
                init, make_chunk_post=None):
    start0 = (lo // 8) * 8
    nch = (hi - start0 + C - 1) // C
    nchm1 = jnp.maximum(nch - 1, 0)

    def cbase(k):
        return jnp.minimum(start0 + jnp.minimum(k, nchm1) * C, N - C)

    _issue(x_hbm, batch_hbm, xbufs[0], ibufs[0], sems[0], sems[1], cbase(0))
    _issue(x_hbm, batch_hbm, xbufs[1], ibufs[1], sems[2], sems[3], cbase(1))

    def pair_body(p, carry):
        for half in (0, 1):
            k = 2 * p + half
            xb, ib = xbufs[half], ibufs[half]
            sx, sb = sems[2 * half], sems[2 * half + 1]
            _wait(x_hbm, batch_hbm, xb, ib, sx, sb)
            base = cbase(k)
            i_lo = jnp.maximum(jnp.maximum(lo, start0 + k * C) - base, 0)
            i_hi = jnp.clip(hi - base, 0, C)
            i_hi = jnp.where(k < nch, i_hi, 0)
            carry = make_row_body(xb, ib)(i_lo, i_hi, carry)
            if make_chunk_post is not None:
                carry = make_chunk_post(xb, ib)(i_lo, i_hi, carry)
            _issue(x_hbm, batch_hbm, xb, ib, sx, sb, cbase(k + 2))
        return carry

    carry = lax.fori_loop(0, (nch + 1) // 2, pair_body, init)
    for half in (0, 1):
        _wait(x_hbm, batch_hbm, xbufs[half], ibufs[half],
              sems[2 * half], sems[2 * half + 1])
    return carry


@functools.partial(
    pl.kernel,
    mesh=_mesh,
    out_type=[
        jax.ShapeDtypeStruct((G * (D + 16),), jnp.float32),
        jax.ShapeDtypeStruct((NB * 16,), jnp.int32),
    ],
    scratch_types=[
        pltpu.VMEM((C, D), jnp.float32),
        pltpu.VMEM((C, D), jnp.float32),
        pltpu.VMEM((C + 16,), jnp.int32),
        pltpu.VMEM((C + 16,), jnp.int32),
        pltpu.VMEM((SB,), jnp.int32),
        pltpu.VMEM((SB,), jnp.int32),
        pltpu.VMEM((GPT * (D + 16),), jnp.float32),
    ] + [pltpu.SemaphoreType.DMA] * 6,
    compiler_params=pltpu.CompilerParams(needs_layout_passes=False),
)
def _sc_pass1(x_hbm, batch_hbm, sums_out, bounds_out,
              xb0, xb1, ib0, ib1, sb0, sb1, acc,
              s0, s1, s2, s3, s4, s5):
    cid = lax.axis_index("c")
    sid = lax.axis_index("s")
    wid = sid * NC + cid
    g0 = GPT * wid

    z = jnp.zeros((16,), jnp.float32)
    for v in range(GPT * (DV + 1)):
        acc[pl.ds(v * 16, 16)] = z

    lo, hi = _row_bounds(batch_hbm, (sb0, sb1), (s4, s5), wid)
    one = jnp.ones((16,), jnp.float32)

    def make_row_body(xb, ib):
        def proc(i_lo, i_hi, carry):
            def cond_fn(c):
                return c[0] < i_hi

            def body_fn(c):
                i, gprev = c[0], c[1]
                g = ib[pl.ds(i, 16)][0]

                def flush_br(ops):
                    o = (gprev - g0) * (D + 16)
                    for j in range(DV):
                        acc[pl.ds(o + j * 16, 16)] = ops[1 + j]
                    acc[pl.ds(o + D, 16)] = ops[0]
                    return (z,) * (DV + 1)

                def keep_br(ops):
                    return ops

                cur = lax.cond(
                    jnp.logical_and(g != gprev, gprev >= 0),
                    flush_br, keep_br, c[2:])
                end = _run_end(ib, i, g, i_hi)

                def inner(ii, ic):
                    new = tuple(
                        a + xb[ii, pl.ds(j * 16, 16)]
                        for j, a in enumerate(ic[1:]))
                    return (ic[0] + one,) + new

                cur = lax.fori_loop(i, end, inner, cur)
                return (end, g) + cur

            return lax.while_loop(cond_fn, body_fn, (i_lo,) + carry)[1:]

        return proc

    init = (jnp.int32(-1), z) + (z,) * DV
    carry = _chunk_loop(x_hbm, batch_hbm, (xb0, xb1), (ib0, ib1),
                        (s0, s1, s2, s3), lo, hi, make_row_body, init)
    gprev, cnt = carry[0], carry[1]
    accs = carry[2:]

    @pl.when(gprev >= 0)
    def _final_flush():
        o = (gprev - g0) * (D + 16)
        for j in range(DV):
            acc[pl.ds(o + j * 16, 16)] = accs[j]
        acc[pl.ds(o + D, 16)] = cnt

    pltpu.sync_copy(
        acc, sums_out.at[pl.ds(wid * GPT * (D + 16), GPT * (D + 16))])
    ib0[pl.ds(0, 16)] = jnp.full((16,), lo, jnp.int32)
    pltpu.sync_copy(ib0.at[pl.ds(0, 16)], bounds_out.at[pl.ds(wid * 16, 16)])

    @pl.when(wid == NW - 1)
    def _last_bound():
        ib1[pl.ds(0, 16)] = jnp.full((16,), hi, jnp.int32)
        pltpu.sync_copy(ib1.at[pl.ds(0, 16)],
                        bounds_out.at[pl.ds(NW * 16, 16)])


def _tc_ctx(sums_ref, w_ref, b_ref, ctx_ref):
    sums = sums_ref[:, :D]
    counts = jnp.maximum(sums_ref[:, D:D + 1], 1.0)
    mean = sums / counts
    ctx = lax.dot_general(mean, w_ref[...], (((1,), (1,)), ((), ())),
                          preferred_element_type=jnp.float32)
    ctx_ref[...] = jnp.tanh(ctx + b_ref[0, :][None, :])


@functools.partial(
    pl.kernel,
    mesh=_mesh,
    out_type=jax.ShapeDtypeStruct((G * D,), jnp.float32),
    scratch_types=[
        pltpu.VMEM((C, D), jnp.float32),
        pltpu.VMEM((C, D), jnp.float32),
        pltpu.VMEM((C + 16,), jnp.int32),
        pltpu.VMEM((C + 16,), jnp.int32),
        pltpu.VMEM((32,), jnp.int32),
        pltpu.VMEM((GPT, D), jnp.float32),
        pltpu.VMEM((GPT * D,), jnp.float32),
        pltpu.VMEM((C * 16,), jnp.float32),
        pltpu.VMEM((C,), jnp.float32),
    ] + [pltpu.SemaphoreType.DMA] * 4,
    compiler_params=pltpu.CompilerParams(needs_layout_passes=False),
)
def _sc_pass2(x_hbm, batch_hbm, ctx_hbm, bounds_hbm, out_hbm,
              xb0, xb1, ib0, ib1, bb, ctxb, acc, dbuf, svbuf,
              s0, s1, s2, s3):
    cid = lax.axis_index("c")
    sid = lax.axis_index("s")
    wid = sid * NC + cid
    g0 = GPT * wid

    z = jnp.zeros((16,), jnp.float32)
    for v in range(GPT * DV):
        acc[pl.ds(v * 16, 16)] = z

    pltpu.sync_copy(ctx_hbm.at[pl.ds(g0, GPT)], ctxb)
    pltpu.sync_copy(bounds_hbm.at[pl.ds(wid * 16, 32)], bb)
    lo = bb[pl.ds(0, 16)][0]
    hi = bb[pl.ds(16, 16)][0]
    iota = lax.iota(jnp.int32, 16)

    def make_row_body(xb, ib):
        def proc(i_lo, i_hi, carry):
            def cond_fn(c):
                return c[0] < i_hi

            def body_fn(c):
                i = c[0]
                g = ib[pl.ds(i, 16)][0]
                gi = g - g0
                ctxv = [ctxb[gi, pl.ds(j * 16, 16)] for j in range(DV)]
                end = _run_end(ib, i, g, i_hi)

                @plsc.parallel_loop(i, end, unroll=2)
                def inner(ii):
                    d = [z, z, z, z]
                    for j in range(DV):
                        d[j & 3] = d[j & 3] + xb[ii, pl.ds(j * 16, 16)] * ctxv[j]
                    dbuf[pl.ds(ii * 16, 16)] = (d[0] + d[1]) + (d[2] + d[3])

                return (end,)

            lax.while_loop(cond_fn, body_fn, (i_lo,))
            return carry

        return proc

    def make_chunk_post(xb, ib):
        def chunk_post(i_lo, i_hi, carry):
            for r0 in range(0, C, 16):
                s = z
                for cc in range(16):
                    s = s + plsc.load_gather(
                        dbuf, [(r0 + iota) * 16 + cc])
                svbuf[pl.ds(r0, 16)] = 1.0 / (1.0 + jnp.exp(-s))

            def cond_fn(c):
                return c[0] < i_hi

            def body_fn(c):
                i, gc = c[0], c[1]
                g = ib[pl.ds(i, 16)][0]

                def flush_br(ops):
                    o = (gc - g0) * D
                    for j in range(DV):
                        acc[pl.ds(o + j * 16, 16)] = ops[j]
                    return (z,) * DV

                def keep_br(ops):
                    return ops

                cur = lax.cond(
                    jnp.logical_and(g != gc, gc >= 0),
                    flush_br, keep_br, c[2:])
                end = _run_end(ib, i, g, i_hi)

                @plsc.parallel_loop(i, end, unroll=2, carry=cur)
                def cur(ii, ic):
                    sv = plsc.load_gather(
                        svbuf, [jnp.full((16,), ii, jnp.int32)])
                    return tuple(
                        a + sv * xb[ii, pl.ds(j * 16, 16)]
                        for j, a in enumerate(ic))

                return (end, g) + cur

            return lax.while_loop(cond_fn, body_fn, (i_lo,) + carry)[1:]

        return chunk_post

    init = (jnp.int32(-1),) + (z,) * DV
    carry = _chunk_loop(x_hbm, batch_hbm, (xb0, xb1), (ib0, ib1),
                        (s0, s1, s2, s3), lo, hi, make_row_body,
                        init, make_chunk_post)
    gprev = carry[0]
    accs = carry[1:]

    @pl.when(gprev >= 0)
    def _final_flush():
        o = (gprev - g0) * D
        for j in range(DV):
            acc[pl.ds(o + j * 16, 16)] = accs[j]

    pltpu.sync_copy(acc, out_hbm.at[pl.ds(wid * GPT * D, GPT * D)])


@jax.jit
def kernel(x, batch, W, b):
    batch = batch.astype(jnp.int32)
    sums_cnt, bounds = _sc_pass1(x, batch)
    ctx = pl.pallas_call(
        _tc_ctx,
        out_shape=jax.ShapeDtypeStruct((G, D), jnp.float32),
    )(sums_cnt.reshape(G, D + 16), W, b.reshape(1, D))
    return _sc_pass2(x, batch, ctx, bounds).reshape(G, D)


# speedup vs baseline: 1.3621x; 1.0211x over previous
import functools

import jax
import jax.numpy as jnp
from jax import lax
from jax.experimental import pallas as pl
from jax.experimental.pallas import tpu as pltpu
from jax.experimental.pallas import tpu_sc as plsc

N = 100000
D = 512
G = 512
C = 96
NC = 2
NS = 16
NW = NC * NS
GPT = G // NW
DV = D // 16
SB = 4000
NSB = N // SB
NB = NW + 16

_mesh = plsc.VectorSubcoreMesh(core_axis_name="c", subcore_axis_name="s")
_Z = None


def _lane_sum(v):
    iota = lax.iota(jnp.int32, 16)
    for shift in (8, 4, 2, 1):
        idx = jnp.bitwise_and(iota + shift, 15)
        v = v + jnp.take_along_axis(v, idx, 0, mode="promise_in_bounds")
    return v


def _row_bounds(batch_hbm, sbufs, sems, wid):
    lo_t = GPT * wid
    hi_t = lo_t + GPT
    clo = jnp.zeros((16,), jnp.int32)
    chi = jnp.zeros((16,), jnp.int32)
    pltpu.async_copy(batch_hbm.at[pl.ds(0, SB)], sbufs[0], sems[0])
    for c in range(NSB):
        b = c & 1
        pltpu.make_async_copy(
            batch_hbm.at[pl.ds(0, SB)], sbufs[b], sems[b]).wait()
        if c + 1 < NSB:
            pltpu.async_copy(
                batch_hbm.at[pl.ds((c + 1) * SB, SB)],
                sbufs[1 - b], sems[1 - b])
        sbuf = sbufs[b]

        def body(j, carry):
            cl, ch = carry
            bv = sbuf[pl.ds(j * 16, 16)]
            cl = cl + jnp.where(bv < lo_t, 1, 0)
            ch = ch + jnp.where(bv < hi_t, 1, 0)
            return (cl, ch)

        clo, chi = lax.fori_loop(0, SB // 16, body, (clo, chi))
    return _lane_sum(clo)[0], _lane_sum(chi)[0]


def _run_end(ib, i, g, i_hi):
    iota = lax.iota(jnp.int32, 16)

    def cond_fn(c):
        return c[1] < 0

    def body_fn(c):
        j = c[0]
        bv = ib[pl.ds(j, 16)]
        m = jnp.logical_and(bv == g, (j + iota) < i_hi)
        cnt = plsc.all_reduce_population_count(m)[0]
        end = jnp.where(cnt < 16, j + cnt, -1)
        return (j + 16, end)

    return lax.while_loop(cond_fn, body_fn, (i, jnp.int32(-1)))[1]


def _issue(x_hbm, batch_hbm, xb, ib, sx, sb, base):
    pltpu.async_copy(x_hbm.at[pl.ds(base, C)], xb, sx)
    pltpu.async_copy(batch_hbm.at[pl.ds(base, C)], ib.at[pl.ds(0, C)], sb)


def _wait(x_hbm, batch_hbm, xb, ib, sx, sb):
    pltpu.make_async_copy(x_hbm.at[pl.ds(0, C)], xb, sx).wait()
    pltpu.make_async_copy(
        batch_hbm.at[pl.ds(0, C)], ib.at[pl.ds(0, C)], sb).wait()


def _chunk_loop(x_hbm, batch_hbm, xbufs, ibufs, sems, lo, hi, make_row_body,
                init, make_chunk_post=None):
    start0 = (lo // 8) * 8
    nch = (hi - start0 + C - 1) // C
    nchm1 = jnp.maximum(nch - 1, 0)

    def cbase(k):
        return jnp.minimum(start0 + jnp.minimum(k, nchm1) * C, N - C)

    _issue(x_hbm, batch_hbm, xbufs[0], ibufs[0], sems[0], sems[1], cbase(0))
    _issue(x_hbm, batch_hbm, xbufs[1], ibufs[1], sems[2], sems[3], cbase(1))

    def pair_body(p, carry):
        for half in (0, 1):
            k = 2 * p + half
            xb, ib = xbufs[half], ibufs[half]
            sx, sb = sems[2 * half], sems[2 * half + 1]
            _wait(x_hbm, batch_hbm, xb, ib, sx, sb)
            base = cbase(k)
            i_lo = jnp.maximum(jnp.maximum(lo, start0 + k * C) - base, 0)
            i_hi = jnp.clip(hi - base, 0, C)
            i_hi = jnp.where(k < nch, i_hi, 0)
            carry = make_row_body(xb, ib)(i_lo, i_hi, carry)
            if make_chunk_post is not None:
                carry = make_chunk_post(xb, ib)(i_lo, i_hi, carry)
            _issue(x_hbm, batch_hbm, xb, ib, sx, sb, cbase(k + 2))
        return carry

    carry = lax.fori_loop(0, (nch + 1) // 2, pair_body, init)
    for half in (0, 1):
        _wait(x_hbm, batch_hbm, xbufs[half], ibufs[half],
              sems[2 * half], sems[2 * half + 1])
    return carry


@functools.partial(
    pl.kernel,
    mesh=_mesh,
    out_type=[
        jax.ShapeDtypeStruct((G * (D + 16),), jnp.float32),
        jax.ShapeDtypeStruct((NB * 16,), jnp.int32),
    ],
    scratch_types=[
        pltpu.VMEM((C, D), jnp.float32),
        pltpu.VMEM((C, D), jnp.float32),
        pltpu.VMEM((C + 16,), jnp.int32),
        pltpu.VMEM((C + 16,), jnp.int32),
        pltpu.VMEM((SB,), jnp.int32),
        pltpu.VMEM((SB,), jnp.int32),
        pltpu.VMEM((GPT * (D + 16),), jnp.float32),
    ] + [pltpu.SemaphoreType.DMA] * 6,
    compiler_params=pltpu.CompilerParams(needs_layout_passes=False),
)
def _sc_pass1(x_hbm, batch_hbm, sums_out, bounds_out,
              xb0, xb1, ib0, ib1, sb0, sb1, acc,
              s0, s1, s2, s3, s4, s5):
    cid = lax.axis_index("c")
    sid = lax.axis_index("s")
    wid = sid * NC + cid
    g0 = GPT * wid

    z = jnp.zeros((16,), jnp.float32)
    for v in range(GPT * (DV + 1)):
        acc[pl.ds(v * 16, 16)] = z

    lo, hi = _row_bounds(batch_hbm, (sb0, sb1), (s4, s5), wid)
    one = jnp.ones((16,), jnp.float32)

    def make_row_body(xb, ib):
        def proc(i_lo, i_hi, carry):
            def cond_fn(c):
                return c[0] < i_hi

            def body_fn(c):
                i, gprev = c[0], c[1]
                g = ib[pl.ds(i, 16)][0]

                def flush_br(ops):
                    o = (gprev - g0) * (D + 16)
                    for j in range(DV):
                        acc[pl.ds(o + j * 16, 16)] = ops[1 + j]
                    acc[pl.ds(o + D, 16)] = ops[0]
                    return (z,) * (DV + 1)

                def keep_br(ops):
                    return ops

                cur = lax.cond(
                    jnp.logical_and(g != gprev, gprev >= 0),
                    flush_br, keep_br, c[2:])
                end = _run_end(ib, i, g, i_hi)

                def inner(ii, ic):
                    new = tuple(
                        a + xb[ii, pl.ds(j * 16, 16)]
                        for j, a in enumerate(ic[1:]))
                    return (ic[0] + one,) + new

                cur = lax.fori_loop(i, end, inner, cur)
                return (end, g) + cur

            return lax.while_loop(cond_fn, body_fn, (i_lo,) + carry)[1:]

        return proc

    init = (jnp.int32(-1), z) + (z,) * DV
    carry = _chunk_loop(x_hbm, batch_hbm, (xb0, xb1), (ib0, ib1),
                        (s0, s1, s2, s3), lo, hi, make_row_body, init)
    gprev, cnt = carry[0], carry[1]
    accs = carry[2:]

    @pl.when(gprev >= 0)
    def _final_flush():
        o = (gprev - g0) * (D + 16)
        for j in range(DV):
            acc[pl.ds(o + j * 16, 16)] = accs[j]
        acc[pl.ds(o + D, 16)] = cnt

    pltpu.sync_copy(
        acc, sums_out.at[pl.ds(wid * GPT * (D + 16), GPT * (D + 16))])
    ib0[pl.ds(0, 16)] = jnp.full((16,), lo, jnp.int32)
    pltpu.sync_copy(ib0.at[pl.ds(0, 16)], bounds_out.at[pl.ds(wid * 16, 16)])

    @pl.when(wid == NW - 1)
    def _last_bound():
        ib1[pl.ds(0, 16)] = jnp.full((16,), hi, jnp.int32)
        pltpu.sync_copy(ib1.at[pl.ds(0, 16)],
                        bounds_out.at[pl.ds(NW * 16, 16)])


def _tc_ctx(sums_ref, w_ref, b_ref, ctx_ref):
    sums = sums_ref[:, :D]
    counts = jnp.maximum(sums_ref[:, D:D + 1], 1.0)
    mean = sums / counts
    ctx = lax.dot_general(mean, w_ref[...], (((1,), (1,)), ((), ())),
                          preferred_element_type=jnp.float32)
    ctx_ref[...] = jnp.tanh(ctx + b_ref[0, :][None, :])


@functools.partial(
    pl.kernel,
    mesh=_mesh,
    out_type=jax.ShapeDtypeStruct((G * D,), jnp.float32),
    scratch_types=[
        pltpu.VMEM((C, D), jnp.float32),
        pltpu.VMEM((C, D), jnp.float32),
        pltpu.VMEM((C + 16,), jnp.int32),
        pltpu.VMEM((C + 16,), jnp.int32),
        pltpu.VMEM((32,), jnp.int32),
        pltpu.VMEM((GPT, D), jnp.float32),
        pltpu.VMEM((GPT * D,), jnp.float32),
        pltpu.VMEM((C * 16,), jnp.float32),
        pltpu.VMEM((C,), jnp.float32),
    ] + [pltpu.SemaphoreType.DMA] * 4,
    compiler_params=pltpu.CompilerParams(needs_layout_passes=False),
)
def _sc_pass2(x_hbm, batch_hbm, ctx_hbm, bounds_hbm, out_hbm,
              xb0, xb1, ib0, ib1, bb, ctxb, acc, dbuf, svbuf,
              s0, s1, s2, s3):
    cid = lax.axis_index("c")
    sid = lax.axis_index("s")
    wid = sid * NC + cid
    g0 = GPT * wid

    z = jnp.zeros((16,), jnp.float32)
    for v in range(GPT * DV):
        acc[pl.ds(v * 16, 16)] = z

    pltpu.sync_copy(ctx_hbm.at[pl.ds(g0, GPT)], ctxb)
    pltpu.sync_copy(bounds_hbm.at[pl.ds(wid * 16, 32)], bb)
    lo = bb[pl.ds(0, 16)][0]
    hi = bb[pl.ds(16, 16)][0]
    iota = lax.iota(jnp.int32, 16)

    def make_row_body(xb, ib):
        def proc(i_lo, i_hi, carry):
            def cond_fn(c):
                return c[0] < i_hi

            def body_fn(c):
                i = c[0]
                g = ib[pl.ds(i, 16)][0]
                gi = g - g0
                ctxv = [ctxb[gi, pl.ds(j * 16, 16)] for j in range(DV)]
                end = _run_end(ib, i, g, i_hi)

                @plsc.parallel_loop(i, end, unroll=2)
                def inner(ii):
                    d = [z, z, z, z]
                    for j in range(DV):
                        d[j & 3] = d[j & 3] + xb[ii, pl.ds(j * 16, 16)] * ctxv[j]
                    dbuf[pl.ds(ii * 16, 16)] = (d[0] + d[1]) + (d[2] + d[3])

                return (end,)

            lax.while_loop(cond_fn, body_fn, (i_lo,))
            return carry

        return proc

    def make_chunk_post(xb, ib):
        def chunk_post(i_lo, i_hi, carry):
            for r0 in range(0, C, 16):
                s = z
                for cc in range(16):
                    s = s + plsc.load_gather(
                        dbuf, [(r0 + iota) * 16 + cc])
                svbuf[pl.ds(r0, 16)] = 1.0 / (1.0 + jnp.exp(-s))

            def cond_fn(c):
                return c[0] < i_hi

            def body_fn(c):
                i, gc = c[0], c[1]
                g = ib[pl.ds(i, 16)][0]

                def flush_br(ops):
                    o = (gc - g0) * D
                    for j in range(DV):
                        acc[pl.ds(o + j * 16, 16)] = ops[j]
                    return (z,) * DV

                def keep_br(ops):
                    return ops

                cur = lax.cond(
                    jnp.logical_and(g != gc, gc >= 0),
                    flush_br, keep_br, c[2:])
                end = _run_end(ib, i, g, i_hi)

                @plsc.parallel_loop(i, end, unroll=2, carry=cur)
                def cur(ii, ic):
                    sv = plsc.load_gather(
                        svbuf, [jnp.full((16,), ii, jnp.int32)])
                    return tuple(
                        a + sv * xb[ii, pl.ds(j * 16, 16)]
                        for j, a in enumerate(ic))

                return (end, g) + cur

            return lax.while_loop(cond_fn, body_fn, (i_lo,) + carry)[1:]

        return chunk_post

    init = (jnp.int32(-1),) + (z,) * DV
    carry = _chunk_loop(x_hbm, batch_hbm, (xb0, xb1), (ib0, ib1),
                        (s0, s1, s2, s3), lo, hi, make_row_body,
                        init, make_chunk_post)
    gprev = carry[0]
    accs = carry[1:]

    @pl.when(gprev >= 0)
    def _final_flush():
        o = (gprev - g0) * D
        for j in range(DV):
            acc[pl.ds(o + j * 16, 16)] = accs[j]

    pltpu.sync_copy(acc, out_hbm.at[pl.ds(wid * GPT * D, GPT * D)])


@jax.jit
def kernel(x, batch, W, b):
    batch = batch.astype(jnp.int32)
    sums_cnt, bounds = _sc_pass1(x, batch)
    ctx = pl.pallas_call(
        _tc_ctx,
        out_shape=jax.ShapeDtypeStruct((G, D), jnp.float32),
    )(sums_cnt.reshape(G, D + 16), W, b.reshape(1, D))
    return _sc_pass2(x, batch, ctx, bounds).reshape(G, D)
